# baseline (speedup 1.0000x reference)
import jax
import jax.numpy as jnp
from jax.experimental import pallas as pl
from jax.experimental.pallas import tpu as pltpu
from jax.experimental.pallas import tpu_sc as plsc

EMBED_DIM = 32
WINDOW = 128


def _gather_sc(table, indices):
    num_indices = indices.shape[0]
    idx2d = indices.reshape(num_indices // WINDOW, WINDOW)
    mesh = plsc.VectorSubcoreMesh(core_axis_name="core", subcore_axis_name="subcore")

    @pl.kernel(
        out_type=jax.ShapeDtypeStruct((num_indices, EMBED_DIM), table.dtype),
        mesh=mesh,
        compiler_params=pltpu.CompilerParams(use_tc_tiling_on_sc=False),
    )
    def kern(x_hbm, i_hbm, o_hbm):
        def body(i_vmem, o_vmem):
            pltpu.sync_copy(x_hbm.at[i_vmem.at[0]], o_vmem)

        pltpu.emit_pipeline(
            body,
            grid=(num_indices // WINDOW,),
            in_specs=[pl.BlockSpec((1, WINDOW), index_map=lambda i: (i, 0))],
            out_specs=[pl.BlockSpec((WINDOW, EMBED_DIM), index_map=lambda i: (i, 0))],
            core_axis_name=("core", "subcore"),
            dimension_semantics=(pltpu.PARALLEL,),
        )(i_hbm, o_hbm)

    return kern(table, idx2d)


def _retile_tc(out_lin, batch, hist):
    nchunk = batch // 2048
    x4 = out_lin.reshape(hist, nchunk, 512, 128)

    def body(x_ref, o_ref):
        x = x_ref[0, 0]
        y = x.T.reshape(4, 32, 512)
        for r in range(4):
            o_ref[0, :, pl.ds(r * 512, 512)] = y[r]

    return pl.pallas_call(
        body,
        grid=(hist, nchunk),
        in_specs=[pl.BlockSpec((1, 1, 512, 128), lambda h, s: (h, s, 0, 0))],
        out_specs=pl.BlockSpec((1, EMBED_DIM, 2048), lambda h, s: (h, 0, s)),
        out_shape=jax.ShapeDtypeStruct((hist, EMBED_DIM, batch), jnp.float32),
    )(x4)


def kernel(tokens, table):
    batch, hist = tokens.shape
    tt = jnp.transpose(tokens).astype(jnp.int32)
    idx = jnp.swapaxes(tt.reshape(hist, batch // 2048, 4, 512), 2, 3)
    out_lin = _gather_sc(table, idx.reshape(batch * hist))
    out_t = _retile_tc(out_lin, batch, hist)
    return jnp.transpose(out_t, (2, 0, 1))
